# baseline (speedup 1.0000x reference)
import jax
import jax.numpy as jnp
from jax import lax
from jax.experimental import pallas as pl
from jax.experimental.pallas import tpu as pltpu
from jax.experimental.pallas import tpu_sc as plsc

N = 20000
NCLS = 81
NPAD = 20480
KEEP = 14000
KPAD = 14080
TILES = 16
CHUNK = NPAD // TILES
VPT = CHUNK // 16
KCH = KPAD // TILES
KV = KCH // 16
BINS = 256
PASSES = 4
RBLK = 2048


def _tc_body(x_ref, rois_ref, bb_ref, im_ref, o_ref, x1_ref, y1_ref,
             x2_ref, y2_ref):
  i = pl.program_id(0)
  x = x_ref[...]
  col = lax.broadcasted_iota(jnp.int32, (RBLK, NCLS), 1)
  m = jnp.max(jnp.where(col == 0, -jnp.inf, x), axis=1)
  b = lax.bitcast_convert_type(m, jnp.uint32)
  flip = jnp.where((b >> 31) != 0, jnp.uint32(0xFFFFFFFF), jnp.uint32(0x80000000))
  key = ~(b ^ flip)
  row = i * RBLK + lax.broadcasted_iota(jnp.int32, (RBLK,), 0)
  o_ref[...] = jnp.where(row < N, key, jnp.uint32(0xFFFFFFFF))

  x1 = rois_ref[0, :, 1]
  y1 = rois_ref[0, :, 2]
  x2 = rois_ref[0, :, 3]
  y2 = rois_ref[0, :, 4]
  dx = bb_ref[:, 4]
  dy = bb_ref[:, 5]
  dw = bb_ref[:, 6]
  dh = bb_ref[:, 7]
  w = x2 - x1 + 1.0
  h = y2 - y1 + 1.0
  cx = x1 + 0.5 * (w - 1.0)
  cy = y1 + 0.5 * (h - 1.0)
  pcx = dx * w + cx
  pcy = dy * h + cy
  pw = jnp.exp(dw) * w
  ph = jnp.exp(dh) * h
  xmax = im_ref[0, 1] - 1.0
  ymax = im_ref[0, 0] - 1.0
  x1_ref[...] = jnp.clip(pcx - 0.5 * (pw - 1.0), 0.0, xmax)
  y1_ref[...] = jnp.clip(pcy - 0.5 * (ph - 1.0), 0.0, ymax)
  x2_ref[...] = jnp.clip(pcx + 0.5 * (pw - 1.0), 0.0, xmax)
  y2_ref[...] = jnp.clip(pcy + 0.5 * (ph - 1.0), 0.0, ymax)


_col = jax.ShapeDtypeStruct((NPAD,), jnp.float32)
_tc_keys_decode = pl.pallas_call(
    _tc_body,
    grid=(NPAD // RBLK,),
    in_specs=[
        pl.BlockSpec((RBLK, NCLS), lambda i: (i, 0)),
        pl.BlockSpec((1, RBLK, 5), lambda i: (0, i, 0)),
        pl.BlockSpec((RBLK, 8), lambda i: (i, 0)),
        pl.BlockSpec(memory_space=pltpu.SMEM),
    ],
    out_specs=[pl.BlockSpec((RBLK,), lambda i: (i,))] * 5,
    out_shape=(jax.ShapeDtypeStruct((NPAD,), jnp.uint32),
               _col, _col, _col, _col),
)


def _sc_body(keys_hbm, x1_hbm, y1_hbm, x2_hbm, y2_hbm,
             out_hbm, keep_hbm,
             spk0, spv0, spk1, spv1, ghist,
             ck, cv, pos, dg, prl, hist, gh, offs,
             kp, gidx, c1, c2, c3, c4, ob, sem):
  t = lax.axis_index("s")
  base = t * CHUNK
  iota16 = lax.iota(jnp.int32, 16)

  pltpu.sync_copy(keys_hbm.at[pl.ds(base, CHUNK)], ck)

  def init_vals(j, c):
    cv[pl.ds(j * 16, 16)] = base + j * 16 + iota16
    return c
  lax.fori_loop(0, VPT, init_vals, 0, unroll=8)

  for p in range(PASSES):
    shift = 8 * p

    for j in range(BINS // 16):
      hist[pl.ds(j * 16, 16)] = jnp.zeros((16,), jnp.int32)

    def hb(j, c):
      sl = pl.ds(j * 16, 16)
      k = ck[sl]
      d = ((k >> shift) & jnp.uint32(0xFF)).astype(jnp.int32)
      cnt, last = plsc.scan_count(d)
      hv = plsc.load_gather(hist, [d])
      dg[sl] = d
      prl[sl] = hv + cnt - 1
      plsc.addupdate_scatter(hist, [d], cnt, mask=last)
      return c
    lax.fori_loop(0, VPT, hb, 0)

    pltpu.sync_copy(hist, ghist.at[t])
    plsc.subcore_barrier()
    pltpu.sync_copy(ghist, gh)

    def pref(g, carry):
      col = jnp.zeros((16,), jnp.int32)
      pre = jnp.zeros((16,), jnp.int32)
      for r in range(TILES):
        v = gh[r, pl.ds(g * 16, 16)]
        col = col + v
        pre = pre + v * (jnp.int32(r) < t).astype(jnp.int32)
      incl = plsc.cumsum(col)
      offs[pl.ds(g * 16, 16)] = incl - col + carry + pre
      return carry + jnp.sum(col)
    lax.fori_loop(0, BINS // 16, pref, jnp.int32(0))

    def rp(j, c):
      sl = pl.ds(j * 16, 16)
      d = dg[sl]
      pos[sl] = plsc.load_gather(offs, [d]) + prl[sl]
      return c
    lax.fori_loop(0, VPT, rp, 0, unroll=4)

    dk, dv = (spk0, spv0) if p % 2 == 0 else (spk1, spv1)
    d0 = pltpu.async_copy(ck, dk.at[pos], sem)
    d1 = pltpu.async_copy(cv, dv.at[pos], sem)
    d0.wait()
    d1.wait()
    plsc.subcore_barrier()
    if p < PASSES - 1:
      pltpu.sync_copy(dk.at[pl.ds(base, CHUNK)], ck)
      pltpu.sync_copy(dv.at[pl.ds(base, CHUNK)], cv)

  kb = t * KCH
  pltpu.sync_copy(spv1.at[pl.ds(kb, KCH)], kp)
  pltpu.sync_copy(kp, keep_hbm.at[pl.ds(kb, KCH)])

  def gx(j, c):
    sl = pl.ds(j * 16, 16)
    gidx[sl] = jnp.minimum(kp[sl], jnp.int32(N - 1))
    return c
  lax.fori_loop(0, KV, gx, 0, unroll=4)

  cbufs = [c1, c2, c3, c4]
  srcs = [x1_hbm, y1_hbm, x2_hbm, y2_hbm]
  descs = []
  for ci in range(4):
    for w in range(KCH // 88):
      descs.append(pltpu.async_copy(
          srcs[ci].at[gidx.at[pl.ds(w * 88, 88)]],
          cbufs[ci].at[pl.ds(w * 88, 88)], sem))
  for dsc in descs:
    dsc.wait()

  zx = jnp.zeros((16,), jnp.float32)

  def dec(j, c):
    sl = pl.ds(j * 16, 16)
    rows5 = (j * 16 + iota16) * 5
    plsc.store_scatter(ob, [rows5], zx)
    plsc.store_scatter(ob, [rows5 + 1], c1[sl])
    plsc.store_scatter(ob, [rows5 + 2], c2[sl])
    plsc.store_scatter(ob, [rows5 + 3], c3[sl])
    plsc.store_scatter(ob, [rows5 + 4], c4[sl])
    return c
  lax.fori_loop(0, KV, dec, 0, unroll=2)

  pltpu.sync_copy(ob, out_hbm.at[pl.ds(kb * 5, KCH * 5)])


_sc_sort = pl.kernel(
    _sc_body,
    out_type=(jax.ShapeDtypeStruct((KPAD * 5,), jnp.float32),
              jax.ShapeDtypeStruct((KPAD,), jnp.int32)),
    mesh=plsc.VectorSubcoreMesh(
        core_axis_name="c", subcore_axis_name="s", num_cores=1),
    compiler_params=pltpu.CompilerParams(
        needs_layout_passes=False, use_tc_tiling_on_sc=False),
    scratch_types=[
        pltpu.VMEM_SHARED((NPAD,), jnp.uint32),
        pltpu.VMEM_SHARED((NPAD,), jnp.int32),
        pltpu.VMEM_SHARED((NPAD,), jnp.uint32),
        pltpu.VMEM_SHARED((NPAD,), jnp.int32),
        pltpu.VMEM_SHARED((TILES, BINS), jnp.int32),
        pltpu.VMEM((CHUNK,), jnp.uint32),
        pltpu.VMEM((CHUNK,), jnp.int32),
        pltpu.VMEM((CHUNK,), jnp.int32),
        pltpu.VMEM((CHUNK,), jnp.int32),
        pltpu.VMEM((CHUNK,), jnp.int32),
        pltpu.VMEM((BINS,), jnp.int32),
        pltpu.VMEM((TILES, BINS), jnp.int32),
        pltpu.VMEM((BINS,), jnp.int32),
        pltpu.VMEM((KCH,), jnp.int32),
        pltpu.VMEM((KCH,), jnp.int32),
        pltpu.VMEM((KCH,), jnp.float32),
        pltpu.VMEM((KCH,), jnp.float32),
        pltpu.VMEM((KCH,), jnp.float32),
        pltpu.VMEM((KCH,), jnp.float32),
        pltpu.VMEM((KCH * 5,), jnp.float32),
        pltpu.SemaphoreType.DMA,
    ],
)


def kernel(rois, cls_prob, bbox_pred_tensor, im_info):
  keys, x1c, y1c, x2c, y2c = _tc_keys_decode(
      cls_prob, rois, bbox_pred_tensor, im_info)
  outp, keepp = _sc_sort(keys, x1c, y1c, x2c, y2c)
  return outp.reshape(KPAD, 5)[:KEEP], keepp[:KEEP]


# speedup vs baseline: 1.2218x; 1.2218x over previous
import jax
import jax.numpy as jnp
from jax import lax
from jax.experimental import pallas as pl
from jax.experimental.pallas import tpu as pltpu
from jax.experimental.pallas import tpu_sc as plsc

N = 20000
NCLS = 81
NPAD = 20480
KEEP = 14000
KPAD = 14080
TILES = 16
CHUNK = NPAD // TILES
VPT = CHUNK // 16
KCH = KPAD // TILES
KV = KCH // 16
BINS = 256
PASSES = 4
RBLK = 2048


def _tc_keys_body(x_ref, o_ref):
  i = pl.program_id(0)
  x = x_ref[...]
  col = lax.broadcasted_iota(jnp.int32, (RBLK, NCLS), 1)
  m = jnp.max(jnp.where(col == 0, -jnp.inf, x), axis=1)
  b = lax.bitcast_convert_type(m, jnp.uint32)
  flip = jnp.where((b >> 31) != 0, jnp.uint32(0xFFFFFFFF), jnp.uint32(0x80000000))
  key = ~(b ^ flip)
  row = i * RBLK + lax.broadcasted_iota(jnp.int32, (RBLK,), 0)
  o_ref[...] = jnp.where(row < N, key, jnp.uint32(0xFFFFFFFF))


_tc_keys = pl.pallas_call(
    _tc_keys_body,
    grid=(NPAD // RBLK,),
    in_specs=[pl.BlockSpec((RBLK, NCLS), lambda i: (i, 0))],
    out_specs=pl.BlockSpec((RBLK,), lambda i: (i,)),
    out_shape=jax.ShapeDtypeStruct((NPAD,), jnp.uint32),
)


def _sc_body(keys_hbm, roisf_hbm, bbf_hbm, clip_hbm,
             out_hbm, keep_hbm,
             spk0, spv0, spk1, spv1, ghist,
             ck, cv, pos, dg, prl, hist, gh, offs,
             kp, gidx, c0, c1, c2, c3, c4, c5, c6, c7, ob, cl, sem):
  t = lax.axis_index("s")
  base = t * CHUNK
  iota16 = lax.iota(jnp.int32, 16)

  pltpu.sync_copy(keys_hbm.at[pl.ds(base, CHUNK)], ck)

  def init_vals(j, c):
    cv[pl.ds(j * 16, 16)] = base + j * 16 + iota16
    return c
  lax.fori_loop(0, VPT, init_vals, 0, unroll=8)

  for p in range(PASSES):
    shift = 8 * p

    for j in range(BINS // 16):
      hist[pl.ds(j * 16, 16)] = jnp.zeros((16,), jnp.int32)

    def hb(j, c):
      sl = pl.ds(j * 16, 16)
      k = ck[sl]
      d = ((k >> shift) & jnp.uint32(0xFF)).astype(jnp.int32)
      cnt, last = plsc.scan_count(d)
      hv = plsc.load_gather(hist, [d])
      dg[sl] = d
      prl[sl] = hv + cnt - 1
      plsc.addupdate_scatter(hist, [d], cnt, mask=last)
      return c
    lax.fori_loop(0, VPT, hb, 0)

    pltpu.sync_copy(hist, ghist.at[t])
    plsc.subcore_barrier()
    pltpu.sync_copy(ghist, gh)

    def pref(g, carry):
      col = jnp.zeros((16,), jnp.int32)
      pre = jnp.zeros((16,), jnp.int32)
      for r in range(TILES):
        v = gh[r, pl.ds(g * 16, 16)]
        col = col + v
        pre = pre + v * (jnp.int32(r) < t).astype(jnp.int32)
      incl = plsc.cumsum(col)
      offs[pl.ds(g * 16, 16)] = incl - col + carry + pre
      return carry + jnp.sum(col)
    lax.fori_loop(0, BINS // 16, pref, jnp.int32(0))

    def rp(j, c):
      sl = pl.ds(j * 16, 16)
      d = dg[sl]
      pos[sl] = plsc.load_gather(offs, [d]) + prl[sl]
      return c
    lax.fori_loop(0, VPT, rp, 0, unroll=4)

    dk, dv = (spk0, spv0) if p % 2 == 0 else (spk1, spv1)
    d0 = pltpu.async_copy(ck, dk.at[pos], sem)
    d1 = pltpu.async_copy(cv, dv.at[pos], sem)
    d0.wait()
    d1.wait()
    plsc.subcore_barrier()
    if p < PASSES - 1:
      pltpu.sync_copy(dk.at[pl.ds(base, CHUNK)], ck)
      pltpu.sync_copy(dv.at[pl.ds(base, CHUNK)], cv)

  kb = t * KCH
  pltpu.sync_copy(spv1.at[pl.ds(kb, KCH)], kp)
  pltpu.sync_copy(kp, keep_hbm.at[pl.ds(kb, KCH)])

  def gx(j, c):
    sl = pl.ds(j * 16, 16)
    v = jnp.minimum(kp[sl], jnp.int32(N - 1))
    v5 = v * 5
    v8 = v * 8
    gidx[pl.ds(j * 16, 16)] = v5 + 1
    gidx[pl.ds(KCH + j * 16, 16)] = v5 + 2
    gidx[pl.ds(2 * KCH + j * 16, 16)] = v5 + 3
    gidx[pl.ds(3 * KCH + j * 16, 16)] = v5 + 4
    gidx[pl.ds(4 * KCH + j * 16, 16)] = v8 + 4
    gidx[pl.ds(5 * KCH + j * 16, 16)] = v8 + 5
    gidx[pl.ds(6 * KCH + j * 16, 16)] = v8 + 6
    gidx[pl.ds(7 * KCH + j * 16, 16)] = v8 + 7
    return c
  lax.fori_loop(0, KV, gx, 0, unroll=4)

  cbufs = [c0, c1, c2, c3, c4, c5, c6, c7]
  descs = []
  for ci in range(8):
    src = roisf_hbm if ci < 4 else bbf_hbm
    for w in range(KCH // 88):
      descs.append(pltpu.async_copy(
          src.at[gidx.at[pl.ds(ci * KCH + w * 88, 88)]],
          cbufs[ci].at[pl.ds(w * 88, 88)], sem))
  for dsc in descs:
    dsc.wait()

  pltpu.sync_copy(clip_hbm, cl)
  zx = jnp.zeros((16,), jnp.float32)
  xmax = cl[pl.ds(0, 16)]
  ymax = cl[pl.ds(16, 16)]

  def dec(j, c):
    sl = pl.ds(j * 16, 16)
    x1 = c0[sl]
    y1 = c1[sl]
    x2 = c2[sl]
    y2 = c3[sl]
    dx = c4[sl]
    dy = c5[sl]
    dw = c6[sl]
    dh = c7[sl]
    w = x2 - x1 + 1.0
    h = y2 - y1 + 1.0
    cx = x1 + 0.5 * (w - 1.0)
    cy = y1 + 0.5 * (h - 1.0)
    pcx = dx * w + cx
    pcy = dy * h + cy
    pw = jnp.exp(dw) * w
    ph = jnp.exp(dh) * h
    ox1 = jnp.clip(pcx - 0.5 * (pw - 1.0), 0.0, xmax)
    oy1 = jnp.clip(pcy - 0.5 * (ph - 1.0), 0.0, ymax)
    ox2 = jnp.clip(pcx + 0.5 * (pw - 1.0), 0.0, xmax)
    oy2 = jnp.clip(pcy + 0.5 * (ph - 1.0), 0.0, ymax)
    rows5 = (j * 16 + iota16) * 5
    plsc.store_scatter(ob, [rows5], zx)
    plsc.store_scatter(ob, [rows5 + 1], ox1)
    plsc.store_scatter(ob, [rows5 + 2], oy1)
    plsc.store_scatter(ob, [rows5 + 3], ox2)
    plsc.store_scatter(ob, [rows5 + 4], oy2)
    return c
  lax.fori_loop(0, KV, dec, 0, unroll=2)

  pltpu.sync_copy(ob, out_hbm.at[pl.ds(kb * 5, KCH * 5)])


_sc_sort = pl.kernel(
    _sc_body,
    out_type=(jax.ShapeDtypeStruct((KPAD * 5,), jnp.float32),
              jax.ShapeDtypeStruct((KPAD,), jnp.int32)),
    mesh=plsc.VectorSubcoreMesh(
        core_axis_name="c", subcore_axis_name="s", num_cores=1),
    compiler_params=pltpu.CompilerParams(
        needs_layout_passes=False, use_tc_tiling_on_sc=False),
    scratch_types=[
        pltpu.VMEM_SHARED((NPAD,), jnp.uint32),
        pltpu.VMEM_SHARED((NPAD,), jnp.int32),
        pltpu.VMEM_SHARED((NPAD,), jnp.uint32),
        pltpu.VMEM_SHARED((NPAD,), jnp.int32),
        pltpu.VMEM_SHARED((TILES, BINS), jnp.int32),
        pltpu.VMEM((CHUNK,), jnp.uint32),
        pltpu.VMEM((CHUNK,), jnp.int32),
        pltpu.VMEM((CHUNK,), jnp.int32),
        pltpu.VMEM((CHUNK,), jnp.int32),
        pltpu.VMEM((CHUNK,), jnp.int32),
        pltpu.VMEM((BINS,), jnp.int32),
        pltpu.VMEM((TILES, BINS), jnp.int32),
        pltpu.VMEM((BINS,), jnp.int32),
        pltpu.VMEM((KCH,), jnp.int32),
        pltpu.VMEM((8 * KCH,), jnp.int32),
        pltpu.VMEM((KCH,), jnp.float32),
        pltpu.VMEM((KCH,), jnp.float32),
        pltpu.VMEM((KCH,), jnp.float32),
        pltpu.VMEM((KCH,), jnp.float32),
        pltpu.VMEM((KCH,), jnp.float32),
        pltpu.VMEM((KCH,), jnp.float32),
        pltpu.VMEM((KCH,), jnp.float32),
        pltpu.VMEM((KCH,), jnp.float32),
        pltpu.VMEM((KCH * 5,), jnp.float32),
        pltpu.VMEM((32,), jnp.float32),
        pltpu.SemaphoreType.DMA,
    ],
)


def kernel(rois, cls_prob, bbox_pred_tensor, im_info):
  keys = _tc_keys(cls_prob)
  clipv = jnp.concatenate([jnp.full((16,), im_info[0, 1] - 1.0),
                           jnp.full((16,), im_info[0, 0] - 1.0)])
  roisf = rois.reshape(-1)
  bbf = bbox_pred_tensor.reshape(-1)
  outp, keepp = _sc_sort(keys, roisf, bbf, clipv)
  return outp.reshape(KPAD, 5)[:KEEP], keepp[:KEEP]
